# baseline (speedup 1.0000x reference)
import functools

import jax
import jax.numpy as jnp
from jax import lax
from jax.experimental import pallas as pl
from jax.experimental.pallas import tpu as pltpu
from jax.experimental.pallas import tpu_sc as plsc

_B, _C, _T, _H, _W = 4, 64, 10, 32, 32
_Q = _H * _W
_K = _T * _Q
_TEMP = 0.07
_TOPK = 10
_QT = 128
_NQT = _Q // _QT
_BQ = _B * _Q
_CH = 128
_NCH = _K // _CH
_NEG = float("-inf")
_QPW = _BQ // 32
_CAP = 128
_CMXW = 128


def _aff_kernel(q_ref, k_ref, mask_ref, aff_ref, cmx_ref):
    qn = q_ref[0]
    cmx_ref[...] = jnp.full((1, _QT, _CMXW), _NEG, jnp.float32)
    for t in range(_T):
        for jq in range(_Q // _CH):
            kc = k_ref[0, t, :, pl.ds(jq * _CH, _CH)]
            mc = mask_ref[:, pl.ds(jq * _CH, _CH)]
            st = lax.dot_general(
                qn, kc, (((0,), (0,)), ((), ())),
                precision=lax.Precision.DEFAULT,
                preferred_element_type=jnp.float32)
            sm = jnp.where(mc, st / _TEMP, _NEG)
            c = t * (_Q // _CH) + jq
            aff_ref[c, :, :] = sm
            cmx_ref[0, :, pl.ds(c, 1)] = jnp.max(sm, axis=1, keepdims=True)


def _tc_stage(q, k, maskq):
    return pl.pallas_call(
        _aff_kernel,
        grid=(_B, _NQT),
        in_specs=[
            pl.BlockSpec((1, _C, _QT), lambda b, j: (b, 0, j)),
            pl.BlockSpec((1, _T, _C, _Q), lambda b, j: (b, 0, 0, 0)),
            pl.BlockSpec((_QT, _Q), lambda b, j: (j, 0)),
        ],
        out_specs=[
            pl.BlockSpec((_NCH, _QT, _CH), lambda b, j: (0, b * _NQT + j, 0)),
            pl.BlockSpec((1, _QT, _CMXW), lambda b, j: (b * _NQT + j, 0, 0)),
        ],
        out_shape=[
            jax.ShapeDtypeStruct((_NCH, _BQ, _CH), jnp.float32),
            jax.ShapeDtypeStruct((_B * _NQT, _QT, _CMXW), jnp.float32),
        ],
        compiler_params=pltpu.CompilerParams(
            dimension_semantics=("arbitrary", "arbitrary"),
            vmem_limit_bytes=100 * 1024 * 1024,
        ),
    )(q, k, maskq)


def _merge16(ka, va, kb, vb):
    rb = lax.rev(kb, (0,))
    ri = lax.rev(vb, (0,))
    m = ka >= rb
    kc = jnp.where(m, ka, rb)
    vc = jnp.where(m, va, ri)
    kk, vv = plsc.sort_key_val(kc, vc, descending=True)
    return kk, vv


def _sc_kernel(aff, cmx, v0r, out, cmx_v, idxb, chkb, candv, candi,
               gidx, grows, outb, sem0, sem1, sem2, sem3):
    core = lax.axis_index("c")
    sub = lax.axis_index("s")
    wid = sub * 2 + core
    wbase = wid * _QPW

    pltpu.sync_copy(cmx.at[pl.ds(wbase * _CMXW, _QPW * _CMXW)], cmx_v)

    lanes = lax.iota(jnp.int32, 16)
    sel10 = lanes < _TOPK
    zero16 = jnp.zeros((16,), jnp.float32)
    csem = (sem0, sem1)
    gsem = (sem2, sem3)

    def head(i, par):
        qg = wbase + i
        srt_k = jnp.full((16,), _NEG, jnp.float32)
        srt_v = jnp.zeros((16,), jnp.int32)
        for jv in range(_CMXW // 16):
            cv = cmx_v[pl.ds(i * _CMXW + jv * 16, 16)]
            ids = lanes + (jv * 16)
            kk, vv = plsc.sort_key_val(cv, ids, descending=True)
            srt_k, srt_v = _merge16(srt_k, srt_v, kk, vv)
        tau = jnp.max(jnp.where(lanes == 9, srt_k, _NEG))
        m1 = jnp.max(srt_k)
        rows = srt_v * _BQ + qg
        idxb[pl.ds(par * 16, 16)] = rows
        pltpu.async_copy(
            aff.at[idxb.at[pl.ds(par * 16, 16)]], chkb.at[par], csem[par])
        return tau, m1, srt_v

    def mid(i, par, tau, m1, chunk_ids):
        pltpu.make_async_copy(
            aff.at[idxb.at[pl.ds(par * 16, 16)]], chkb.at[par],
            csem[par]).wait()
        for jz in range(_CAP // 16):
            candv[pl.ds(jz * 16, 16)] = jnp.full((16,), _NEG, jnp.float32)
        tauv = jnp.full((16,), tau, jnp.float32)
        offv = jnp.zeros((16,), jnp.int32)
        for r in range(_TOPK):
            cid = jnp.max(jnp.where(lanes == r, chunk_ids, 0))
            basev = jnp.full((16,), cid * _CH, jnp.int32)
            for v in range(_CH // 16):
                x = chkb[par, r, pl.ds(v * 16, 16)]
                msk = x >= tauv
                cs = plsc.cumsum(msk.astype(jnp.int32))
                pos = offv + cs - 1
                plsc.store_scatter(candv, [pos], x, mask=msk)
                kidx = basev + (v * 16) + lanes
                plsc.store_scatter(candi, [pos], kidx, mask=msk)
                offv = offv + plsc.all_reduce_population_count(msk)
        nc = jnp.max(offv)
        nv = (nc + 15) // 16

        def body(jv, carry):
            sk, sv = carry
            cvals = candv[pl.ds(jv * 16, 16)]
            cidx = candi[pl.ds(jv * 16, 16)]
            kk, vv = plsc.sort_key_val(cvals, cidx, descending=True)
            return _merge16(sk, sv, kk, vv)

        sk, sv = lax.fori_loop(
            0, nv, body,
            (jnp.full((16,), _NEG, jnp.float32), jnp.zeros((16,), jnp.int32)))
        w = jnp.where(sel10, jnp.exp(sk - m1), 0.0)
        n = jnp.sum(w)
        wn = w / n
        gidx[pl.ds(par * 16, 16)] = sv
        pltpu.async_copy(
            v0r.at[gidx.at[pl.ds(par * 16, 16)]], grows.at[par], gsem[par])
        return wn

    def tail(i, par, wn):
        pltpu.make_async_copy(
            v0r.at[gidx.at[pl.ds(par * 16, 16)]], grows.at[par],
            gsem[par]).wait()
        acc = [zero16, zero16, zero16, zero16]
        for r in range(_TOPK):
            wr = jnp.max(jnp.where(lanes == r, wn, 0.0))
            wv = jnp.full((16,), wr, jnp.float32)
            for c4 in range(4):
                acc[c4] = acc[c4] + grows[par, r, pl.ds(c4 * 16, 16)] * wv
        for c4 in range(4):
            outb[pl.ds(i * _C + c4 * 16, 16)] = acc[c4]

    t0, m0, i0 = head(0, 0)
    t1, m1_1, i1 = head(1, 1)
    wn0 = mid(0, 0, t0, m0, i0)

    def run(ii, carry):
        tau_p, m1_p, ids_p, wn_p = carry
        for u in range(2):
            idx = ii * 2 + u
            t_c, m_c, ids_c = head(idx, u)
            wn_c = mid(idx - 1, 1 - u, tau_p, m1_p, ids_p)
            tail(idx - 2, u, wn_p)
            tau_p, m1_p, ids_p, wn_p = t_c, m_c, ids_c, wn_c
        return tau_p, m1_p, ids_p, wn_p

    carry0 = (t1, m1_1, i1, wn0)
    tau_p, m1_p, ids_p, wn_p = lax.fori_loop(1, _QPW // 2, run, carry0)
    wn_l = mid(_QPW - 1, 1, tau_p, m1_p, ids_p)
    tail(_QPW - 2, 0, wn_p)
    tail(_QPW - 1, 1, wn_l)

    pltpu.sync_copy(outb, out.at[pl.ds(wbase * _C, _QPW * _C)])


def _sc_stage(aff_flat, cmx_flat, v0rows):
    mesh = plsc.VectorSubcoreMesh(core_axis_name="c", subcore_axis_name="s")
    kfn = pl.kernel(
        _sc_kernel,
        out_type=jax.ShapeDtypeStruct((_BQ * _C,), jnp.float32),
        mesh=mesh,
        scratch_types=[
            pltpu.VMEM((_QPW * _CMXW,), jnp.float32),
            pltpu.VMEM((32,), jnp.int32),
            pltpu.VMEM((2, 16, _CH), jnp.float32),
            pltpu.VMEM((_CAP,), jnp.float32),
            pltpu.VMEM((_CAP,), jnp.int32),
            pltpu.VMEM((32,), jnp.int32),
            pltpu.VMEM((2, 16, 128), jnp.float32),
            pltpu.VMEM((_QPW * _C,), jnp.float32),
            pltpu.SemaphoreType.DMA,
            pltpu.SemaphoreType.DMA,
            pltpu.SemaphoreType.DMA,
            pltpu.SemaphoreType.DMA,
        ],
        compiler_params=pltpu.CompilerParams(needs_layout_passes=False),
    )
    return kfn(aff_flat, cmx_flat, v0rows)


@functools.partial(jax.jit, static_argnames=())
def kernel(query, key, value, mask):
    qn = query / jnp.maximum(jnp.linalg.norm(query, axis=1, keepdims=True), 1e-12)
    kn = key / jnp.maximum(jnp.linalg.norm(key, axis=1, keepdims=True), 1e-12)
    q = qn.reshape(_B, _C, _Q)
    k = kn.transpose(0, 2, 1, 3, 4).reshape(_B, _T, _C, _Q)
    aff, cmx = _tc_stage(q, k, mask)
    aff_flat = aff.reshape(_NCH * _BQ, _CH)
    cmx_flat = cmx.reshape(_BQ * _CMXW)
    v0rows = jnp.pad(value[0].reshape(_C, _K).T, ((0, 0), (0, 128 - _C)))
    out_flat = _sc_stage(aff_flat, cmx_flat, v0rows)
    out = out_flat.reshape(_B, _Q, _C).transpose(0, 2, 1)
    return out.reshape(_B, _C, _H, _W)


# speedup vs baseline: 1.2764x; 1.2764x over previous
import functools

import jax
import jax.numpy as jnp
from jax import lax
from jax.experimental import pallas as pl
from jax.experimental.pallas import tpu as pltpu
from jax.experimental.pallas import tpu_sc as plsc

_B, _C, _T, _H, _W = 4, 64, 10, 32, 32
_Q = _H * _W
_K = _T * _Q
_TEMP = 0.07
_TOPK = 10
_QT = 128
_NQT = _Q // _QT
_BQ = _B * _Q
_CH = 256
_NCH = _K // _CH
_NEG = float("-inf")
_QPW = _BQ // 32
_CAP = 256
_CMXW = 64


def _aff_kernel(q_ref, k_ref, mask_ref, aff_ref, cmx_ref):
    qn = q_ref[0]
    cmx_ref[...] = jnp.full((1, _QT, _CMXW), _NEG, jnp.float32)
    for t in range(_T):
        for jq in range(_Q // _CH):
            kc = k_ref[0, t, :, pl.ds(jq * _CH, _CH)]
            mc = mask_ref[:, pl.ds(jq * _CH, _CH)]
            st = lax.dot_general(
                qn, kc, (((0,), (0,)), ((), ())),
                precision=lax.Precision.DEFAULT,
                preferred_element_type=jnp.float32)
            sm = jnp.where(mc, st / _TEMP, _NEG)
            c = t * (_Q // _CH) + jq
            aff_ref[c, :, :] = sm
            cmx_ref[0, :, pl.ds(c, 1)] = jnp.max(sm, axis=1, keepdims=True)


def _tc_stage(q, k, maskq):
    return pl.pallas_call(
        _aff_kernel,
        grid=(_B, _NQT),
        in_specs=[
            pl.BlockSpec((1, _C, _QT), lambda b, j: (b, 0, j)),
            pl.BlockSpec((1, _T, _C, _Q), lambda b, j: (b, 0, 0, 0)),
            pl.BlockSpec((_QT, _Q), lambda b, j: (j, 0)),
        ],
        out_specs=[
            pl.BlockSpec((_NCH, _QT, _CH), lambda b, j: (0, b * _NQT + j, 0)),
            pl.BlockSpec((1, _QT, _CMXW), lambda b, j: (b * _NQT + j, 0, 0)),
        ],
        out_shape=[
            jax.ShapeDtypeStruct((_NCH, _BQ, _CH), jnp.float32),
            jax.ShapeDtypeStruct((_B * _NQT, _QT, _CMXW), jnp.float32),
        ],
        compiler_params=pltpu.CompilerParams(
            dimension_semantics=("arbitrary", "arbitrary"),
            vmem_limit_bytes=100 * 1024 * 1024,
        ),
    )(q, k, maskq)


def _merge16(ka, va, kb, vb):
    rb = lax.rev(kb, (0,))
    ri = lax.rev(vb, (0,))
    m = ka >= rb
    kc = jnp.where(m, ka, rb)
    vc = jnp.where(m, va, ri)
    kk, vv = plsc.sort_key_val(kc, vc, descending=True)
    return kk, vv


def _sc_kernel(aff, cmx, v0r, out, cmx_v, idxb, chkb, candv, candi,
               gidx, taub, m1b, wnb, grows, outb, sem0, sem1, sem2, sem3):
    core = lax.axis_index("c")
    sub = lax.axis_index("s")
    wid = sub * 2 + core
    wbase = wid * _QPW

    pltpu.sync_copy(cmx.at[pl.ds(wbase * _CMXW, _QPW * _CMXW)], cmx_v)

    lanes = lax.iota(jnp.int32, 16)
    sel10 = lanes < _TOPK
    zero16 = jnp.zeros((16,), jnp.float32)
    csem = (sem0, sem1)
    gsem = (sem2, sem3)

    def head(i, par):
        qg = wbase + i
        srt_k = jnp.full((16,), _NEG, jnp.float32)
        srt_v = jnp.zeros((16,), jnp.int32)
        for jv in range(_CMXW // 16):
            cv = cmx_v[pl.ds(i * _CMXW + jv * 16, 16)]
            ids = lanes + (jv * 16)
            kk, vv = plsc.sort_key_val(cv, ids, descending=True)
            srt_k, srt_v = _merge16(srt_k, srt_v, kk, vv)
        tau = jnp.max(jnp.where(lanes == 9, srt_k, _NEG))
        m1 = jnp.max(srt_k)
        taub[pl.ds(par * 16, 16)] = jnp.full((16,), tau, jnp.float32)
        m1b[pl.ds(par * 16, 16)] = jnp.full((16,), m1, jnp.float32)
        rows = srt_v * _BQ + qg
        idxb[pl.ds(par * 16, 16)] = rows
        pltpu.async_copy(
            aff.at[idxb.at[pl.ds(par * 16, 16)]], chkb.at[par], csem[par])

    def mid(i, par):
        pltpu.make_async_copy(
            aff.at[idxb.at[pl.ds(par * 16, 16)]], chkb.at[par],
            csem[par]).wait()
        qg = wbase + i
        rows = idxb[pl.ds(par * 16, 16)]
        chunk_ids = lax.shift_right_logical(rows - qg, 12)
        tauv = taub[pl.ds(par * 16, 16)]
        m1v = m1b[pl.ds(par * 16, 16)]
        for jz in range(_CAP // 16):
            candv[pl.ds(jz * 16, 16)] = jnp.full((16,), _NEG, jnp.float32)
        offv = jnp.zeros((16,), jnp.int32)
        for r in range(_TOPK):
            cid = jnp.max(jnp.where(lanes == r, chunk_ids, 0))
            basev = jnp.full((16,), cid * _CH, jnp.int32)
            for v in range(_CH // 16):
                x = chkb[par, r, pl.ds(v * 16, 16)]
                msk = x >= tauv
                cs = plsc.cumsum(msk.astype(jnp.int32))
                pos = offv + cs - 1
                plsc.store_scatter(candv, [pos], x, mask=msk)
                kidx = basev + (v * 16) + lanes
                plsc.store_scatter(candi, [pos], kidx, mask=msk)
                offv = offv + plsc.all_reduce_population_count(msk)
        nc = jnp.max(offv)
        nv = (nc + 15) // 16

        def body(jv, carry):
            sk, sv = carry
            cvals = candv[pl.ds(jv * 16, 16)]
            cidx = candi[pl.ds(jv * 16, 16)]
            kk, vv = plsc.sort_key_val(cvals, cidx, descending=True)
            return _merge16(sk, sv, kk, vv)

        sk, sv = lax.fori_loop(
            0, nv, body,
            (jnp.full((16,), _NEG, jnp.float32), jnp.zeros((16,), jnp.int32)))
        w = jnp.where(sel10, jnp.exp(sk - m1v), 0.0)
        n = jnp.sum(w)
        wnb[pl.ds(par * 16, 16)] = w / n
        gidx[pl.ds(par * 16, 16)] = sv
        pltpu.async_copy(
            v0r.at[gidx.at[pl.ds(par * 16, 16)]], grows.at[par], gsem[par])

    def tail(i, par):
        pltpu.make_async_copy(
            v0r.at[gidx.at[pl.ds(par * 16, 16)]], grows.at[par],
            gsem[par]).wait()
        wn = wnb[pl.ds(par * 16, 16)]
        acc = [zero16, zero16, zero16, zero16]
        for r in range(_TOPK):
            wr = jnp.max(jnp.where(lanes == r, wn, 0.0))
            wv = jnp.full((16,), wr, jnp.float32)
            for c4 in range(4):
                acc[c4] = acc[c4] + grows[par, r, pl.ds(c4 * 16, 16)] * wv
        for c4 in range(4):
            outb[pl.ds(i * _C + c4 * 16, 16)] = acc[c4]

    def run(ii, carry):
        for u in range(2):
            idx = ii * 2 + u

            @pl.when(idx >= 4)
            def _():
                tail(idx - 4, u)

            @pl.when(jnp.logical_and(idx >= 2, idx < _QPW + 2))
            def _():
                mid(idx - 2, u)

            @pl.when(idx < _QPW)
            def _():
                head(idx, u)

        return carry

    lax.fori_loop(0, _QPW // 2 + 2, run, 0)

    pltpu.sync_copy(outb, out.at[pl.ds(wbase * _C, _QPW * _C)])


def _sc_stage(aff_flat, cmx_flat, v0rows):
    mesh = plsc.VectorSubcoreMesh(core_axis_name="c", subcore_axis_name="s")
    kfn = pl.kernel(
        _sc_kernel,
        out_type=jax.ShapeDtypeStruct((_BQ * _C,), jnp.float32),
        mesh=mesh,
        scratch_types=[
            pltpu.VMEM((_QPW * _CMXW,), jnp.float32),
            pltpu.VMEM((32,), jnp.int32),
            pltpu.VMEM((2, 16, _CH), jnp.float32),
            pltpu.VMEM((_CAP,), jnp.float32),
            pltpu.VMEM((_CAP,), jnp.int32),
            pltpu.VMEM((32,), jnp.int32),
            pltpu.VMEM((32,), jnp.float32),
            pltpu.VMEM((32,), jnp.float32),
            pltpu.VMEM((32,), jnp.float32),
            pltpu.VMEM((2, 16, 128), jnp.float32),
            pltpu.VMEM((_QPW * _C,), jnp.float32),
            pltpu.SemaphoreType.DMA,
            pltpu.SemaphoreType.DMA,
            pltpu.SemaphoreType.DMA,
            pltpu.SemaphoreType.DMA,
        ],
        compiler_params=pltpu.CompilerParams(needs_layout_passes=False),
    )
    return kfn(aff_flat, cmx_flat, v0rows)


@functools.partial(jax.jit, static_argnames=())
def kernel(query, key, value, mask):
    qn = query / jnp.maximum(jnp.linalg.norm(query, axis=1, keepdims=True), 1e-12)
    kn = key / jnp.maximum(jnp.linalg.norm(key, axis=1, keepdims=True), 1e-12)
    q = qn.reshape(_B, _C, _Q)
    k = kn.transpose(0, 2, 1, 3, 4).reshape(_B, _T, _C, _Q)
    aff, cmx = _tc_stage(q, k, mask)
    aff_flat = aff.reshape(_NCH * _BQ, _CH)
    cmx_flat = cmx.reshape(_BQ * _CMXW)
    v0rows = jnp.pad(value[0].reshape(_C, _K).T, ((0, 0), (0, 128 - _C)))
    out_flat = _sc_stage(aff_flat, cmx_flat, v0rows)
    out = out_flat.reshape(_B, _Q, _C).transpose(0, 2, 1)
    return out.reshape(_B, _C, _H, _W)
